# baseline (speedup 1.0000x reference)
import functools

import jax
import jax.numpy as jnp
from jax import lax
from jax.experimental import pallas as pl
from jax.experimental.pallas import tpu as pltpu
from jax.experimental.pallas import tpu_sc as plsc

_B, _H, _W, _C = 4, 384, 384, 96
_BS = 16
_HB, _WB = _H // _BS, _W // _BS
_NB = 1152
_ROW = _BS * _C
_NROWS_OUT = _B * _H * _WB
_NROWS_IN = _NB * _BS
_NW = 32
_POS_PER_W = _NB // _NW
_CHUNK_POS = 4
_CHUNK_ROWS = _CHUNK_POS * _BS
_NCHUNK = _POS_PER_W // _CHUNK_POS
_BASE_PAD = 48

_mesh = plsc.VectorSubcoreMesh(core_axis_name="c", subcore_axis_name="s")


@functools.partial(
    pl.kernel,
    mesh=_mesh,
    out_type=jax.ShapeDtypeStruct((_NROWS_OUT, _ROW), jnp.float32),
    scratch_types=[
        pltpu.VMEM((_BASE_PAD,), jnp.int32),
        pltpu.VMEM((_BASE_PAD,), jnp.int32),
        pltpu.VMEM((_CHUNK_ROWS,), jnp.int32),
        pltpu.VMEM((_CHUNK_ROWS, _ROW), jnp.float32),
        pltpu.SemaphoreType.DMA,
    ],
)
def _scatter_kernel(in2d, base_act, base_in, zrows, out, ba_v, bi_v, idx_v,
                    buf, sem):
    wid = lax.axis_index("s") * 2 + lax.axis_index("c")
    row_off = lax.iota(jnp.int32, 16) * _WB

    pltpu.sync_copy(base_act.at[wid], ba_v)
    pltpu.sync_copy(base_in.at[wid], bi_v)
    pltpu.sync_copy(zrows, buf)

    def scatter_chunk(base_v, c):
        g, cl = divmod(c, 4)
        bvec = base_v[pl.ds(g * 16, 16)]
        for j in range(_CHUNK_POS):
            splat = lax.gather(
                bvec, jnp.full((16, 1), cl * 4 + j, jnp.int32),
                dimension_numbers=lax.GatherDimensionNumbers(
                    offset_dims=(), collapsed_slice_dims=(0,),
                    start_index_map=(0,)),
                slice_sizes=(1,),
                mode=lax.GatherScatterMode.PROMISE_IN_BOUNDS)
            idx_v[pl.ds(j * 16, 16)] = splat + row_off
        pltpu.async_copy(buf, out.at[idx_v], sem).wait()

    for c in range(_NCHUNK):
        scatter_chunk(bi_v, c)
    for c in range(_NCHUNK):
        src0 = (wid * _POS_PER_W + c * _CHUNK_POS) * _BS
        pltpu.sync_copy(in2d.at[pl.ds(src0, _CHUNK_ROWS)], buf)
        scatter_chunk(ba_v, c)


def kernel(inputs, bin_counts, active_block_indices):
    del bin_counts
    abi = active_block_indices.astype(jnp.int32)
    bcol, bycol, bxcol = abi[:, 0], abi[:, 1], abi[:, 2]
    base_act = bcol * (_H * _WB) + bycol * (_BS * _WB) + bxcol
    p_act = bcol * (_HB * _WB) + bycol * _WB + bxcol
    occ = jnp.zeros((_B * _HB * _WB,), jnp.bool_).at[p_act].set(
        True, unique_indices=True)
    p_in = jnp.nonzero(~occ, size=_NB, fill_value=0)[0].astype(jnp.int32)
    base_in = ((p_in // (_HB * _WB)) * (_H * _WB)
               + ((p_in % (_HB * _WB)) // _WB) * (_BS * _WB)
               + p_in % _WB)

    pad = ((0, 0), (0, _BASE_PAD - _POS_PER_W))
    base_act2 = jnp.pad(base_act.reshape(_NW, _POS_PER_W), pad)
    base_in2 = jnp.pad(base_in.reshape(_NW, _POS_PER_W), pad)
    in2d = inputs.reshape(_NROWS_IN, _ROW)
    zrows = jnp.zeros((_CHUNK_ROWS, _ROW), jnp.float32)

    out2d = _scatter_kernel(in2d, base_act2, base_in2, zrows)
    return out2d.reshape(_B, _H, _W, _C)
